# baseline (speedup 1.0000x reference)
import jax
import jax.numpy as jnp
from jax import lax
from jax.experimental import pallas as pl
from jax.experimental.pallas import tpu as pltpu
from jax.experimental.pallas import tpu_sc as plsc

_DEPTH = 1000
_N = 4096
_J = 20
_NC = 2
_NS = 16
_NW = _NC * _NS
_IW = _N // _NW
_DC = 200
_NCHUNK = _DEPTH // _DC
_NUNIT = _J * _NCHUNK
_NBUF = 4


def _div5(g):
    return lax.shift_right_logical(g * 52429, 18)


def _body(x_hbm, out_hbm, xs, b0, b1, b2, b3, s0, s1, s2, s3):
    bufs = (b0, b1, b2, b3)
    sems = (s0, s1, s2, s3)
    wid = lax.axis_index("s") * _NC + lax.axis_index("c")

    iota = lax.iota(jnp.int32, 16)
    ones_v = jnp.full((16,), 1.0, jnp.float32)
    zeros_v = jnp.zeros((16,), jnp.float32)

    pltpu.sync_copy(x_hbm.at[wid], xs)
    iota20 = iota * _J

    def _scatter(b, g, val):
        j = _div5(g)
        d0 = (g - j * _NCHUNK) * _DC
        for v in range(_IW // 16):
            col = iota + (v * 16)
            fl = iota20 + (v * 16 * _J + j)
            xi = plsc.load_gather(
                xs, [lax.shift_right_logical(fl, 7), fl & 127]
            )
            u = xi - d0
            m = u.astype(jnp.uint32) < jnp.uint32(_DC)
            row = jnp.where(m, u, 0)
            plsc.store_scatter(bufs[b], [row, col], val, mask=m)

    def _start(b, g):
        j = _div5(g)
        d0 = (g - j * _NCHUNK) * _DC
        dst = out_hbm.at[j, pl.ds(d0, _DC), pl.ds(wid * _IW, _IW)]
        pltpu.async_copy(bufs[b], dst, sems[b])

    def _wait(b):
        dst = out_hbm.at[0, pl.ds(0, _DC), pl.ds(0, _IW)]
        pltpu.make_async_copy(bufs[b], dst, sems[b]).wait()

    def _zero(b):
        def zb(i, carry):
            for v in range(_IW // 16):
                bufs[b][i, pl.ds(v * 16, 16)] = zeros_v
            return carry

        lax.fori_loop(0, _DC, zb, 0)

    for b in range(_NBUF):
        _zero(b)
        _scatter(b, b, ones_v)
        _start(b, b)

    def _step(it, carry):
        for b in range(_NBUF):
            g = it * _NBUF + b
            _wait(b)
            _scatter(b, g - _NBUF, zeros_v)
            _scatter(b, g, ones_v)
            _start(b, g)
        return carry

    lax.fori_loop(1, _NUNIT // _NBUF, _step, 0)

    for b in range(_NBUF):
        _wait(b)


@jax.jit
def _one_hot_sc(x2d):
    mesh = plsc.VectorSubcoreMesh(core_axis_name="c", subcore_axis_name="s")
    f = pl.kernel(
        _body,
        out_type=jax.ShapeDtypeStruct((_J, _DEPTH, _N), jnp.float32),
        mesh=mesh,
        compiler_params=pltpu.CompilerParams(
            needs_layout_passes=False, use_tc_tiling_on_sc=True
        ),
        scratch_types=[pltpu.VMEM((_J, _IW), jnp.int32)]
        + [pltpu.VMEM((_DC, _IW), jnp.float32) for _ in range(_NBUF)]
        + [pltpu.SemaphoreType.DMA for _ in range(_NBUF)],
    )
    return f(x2d)


def kernel(X_in, ones):
    del ones
    x3 = X_in.astype(jnp.int32).reshape(_NW, _J, _IW)
    out = _one_hot_sc(x3)
    return jnp.transpose(out, (2, 1, 0))


# speedup vs baseline: 1.0090x; 1.0090x over previous
import jax
import jax.numpy as jnp
from jax import lax
from jax.experimental import pallas as pl
from jax.experimental.pallas import tpu as pltpu
from jax.experimental.pallas import tpu_sc as plsc

_DEPTH = 1000
_N = 4096
_J = 20
_NC = 2
_NS = 16
_NW = _NC * _NS
_IW = _N // _NW
_DC = 200
_NCHUNK = _DEPTH // _DC
_NUNIT = _J * _NCHUNK
_NBUF = 4


def _div5(g):
    return lax.shift_right_logical(g * 52429, 18)


def _body(x_hbm, out_hbm, xs, b0, b1, b2, b3, s0, s1, s2, s3):
    bufs = (b0, b1, b2, b3)
    sems = (s0, s1, s2, s3)
    wid = lax.axis_index("s") * _NC + lax.axis_index("c")

    iota = lax.iota(jnp.int32, 16)
    ones_v = jnp.full((16,), 1.0, jnp.float32)
    zeros_v = jnp.zeros((16,), jnp.float32)

    pltpu.sync_copy(x_hbm.at[pl.ds(wid * _IW, _IW), :], xs)

    def _scatter(b, g, val):
        j = _div5(g)
        d0 = (g - j * _NCHUNK) * _DC
        jv = jnp.zeros((16,), jnp.int32) + j
        for v in range(_IW // 16):
            col = iota + (v * 16)
            xi = plsc.load_gather(xs, [col, jv])
            u = xi - d0
            m = u.astype(jnp.uint32) < jnp.uint32(_DC)
            row = jnp.where(m, u, 0)
            plsc.store_scatter(bufs[b], [row, col], val, mask=m)

    def _start(b, g):
        j = _div5(g)
        d0 = (g - j * _NCHUNK) * _DC
        dst = out_hbm.at[j, pl.ds(d0, _DC), pl.ds(wid * _IW, _IW)]
        pltpu.async_copy(bufs[b], dst, sems[b])

    def _wait(b):
        dst = out_hbm.at[0, pl.ds(0, _DC), pl.ds(0, _IW)]
        pltpu.make_async_copy(bufs[b], dst, sems[b]).wait()

    def _zero(b):
        def zb(i, carry):
            for v in range(_IW // 16):
                bufs[b][i, pl.ds(v * 16, 16)] = zeros_v
            return carry

        lax.fori_loop(0, _DC, zb, 0)

    for b in range(_NBUF):
        _zero(b)
        _scatter(b, b, ones_v)
        _start(b, b)

    def _step(it, carry):
        for b in range(_NBUF):
            g = it * _NBUF + b
            _wait(b)
            _scatter(b, g - _NBUF, zeros_v)
            _scatter(b, g, ones_v)
            _start(b, g)
        return carry

    lax.fori_loop(1, _NUNIT // _NBUF, _step, 0)

    for b in range(_NBUF):
        _wait(b)


@jax.jit
def _one_hot_sc(x2d):
    mesh = plsc.VectorSubcoreMesh(core_axis_name="c", subcore_axis_name="s")
    f = pl.kernel(
        _body,
        out_type=jax.ShapeDtypeStruct((_J, _DEPTH, _N), jnp.float32),
        mesh=mesh,
        compiler_params=pltpu.CompilerParams(
            needs_layout_passes=False, use_tc_tiling_on_sc=True
        ),
        scratch_types=[pltpu.VMEM((_IW, _J), jnp.int32)]
        + [pltpu.VMEM((_DC, _IW), jnp.float32) for _ in range(_NBUF)]
        + [pltpu.SemaphoreType.DMA for _ in range(_NBUF)],
    )
    return f(x2d)


def kernel(X_in, ones):
    del ones
    out = _one_hot_sc(X_in.astype(jnp.int32))
    return jnp.transpose(out, (2, 1, 0))
